# baseline (speedup 1.0000x reference)
import functools

import jax
import jax.numpy as jnp
from jax.experimental import pallas as pl

BT = 256
NT = 512


def _block_body(x_ref, w_ref, b_ref, g_ref, h_ref, gate_ref):
    c = pl.program_id(1)
    h = jnp.dot(x_ref[...], w_ref[...], preferred_element_type=jnp.float32)
    h = jnp.maximum(h + b_ref[...][None, :], 0.0)
    h_ref[...] = h
    pg = jnp.dot(h, g_ref[...], preferred_element_type=jnp.float32)

    @pl.when(c == 0)
    def _():
        gate_ref[...] = pg

    @pl.when(c != 0)
    def _():
        gate_ref[...] += pg


def _backbone_block(x, w, b, g):
    B, D = x.shape
    N = w.shape[1]
    grid = (B // BT, N // NT)
    return pl.pallas_call(
        _block_body,
        grid=grid,
        in_specs=[
            pl.BlockSpec((BT, D), lambda r, c: (r, 0)),
            pl.BlockSpec((D, NT), lambda r, c: (0, c)),
            pl.BlockSpec((NT,), lambda r, c: (c,)),
            pl.BlockSpec((NT, 1), lambda r, c: (c, 0)),
        ],
        out_specs=[
            pl.BlockSpec((BT, NT), lambda r, c: (r, c)),
            pl.BlockSpec((BT, 1), lambda r, c: (r, 0)),
        ],
        out_shape=[
            jax.ShapeDtypeStruct((B, N), jnp.float32),
            jax.ShapeDtypeStruct((B, 1), jnp.float32),
        ],
    )(x, w, b, g)


def _heads_body(h1_ref, h2_ref, g1_ref, g2_ref,
                we1_ref, be1_ref, we2_ref, be2_ref, w3_ref, b3_ref, y_ref):
    take1 = g1_ref[...] > 0.0
    take2 = jnp.logical_and(jnp.logical_not(take1), g2_ref[...] > 0.0)
    y1 = jnp.dot(h1_ref[...], we1_ref[...],
                 preferred_element_type=jnp.float32) + be1_ref[...][None, :]
    y2 = jnp.dot(h2_ref[...], we2_ref[...],
                 preferred_element_type=jnp.float32) + be2_ref[...][None, :]
    y3 = jnp.dot(h2_ref[...], w3_ref[...],
                 preferred_element_type=jnp.float32) + b3_ref[...][None, :]
    y_ref[...] = jnp.where(take1, y1, jnp.where(take2, y2, y3))


def _heads(h1, h2, gate1, gate2, we1, be1, we2, be2, w3, b3):
    B, D = h1.shape
    N = we1.shape[1]
    grid = (B // BT, N // NT)
    wspec = pl.BlockSpec((D, NT), lambda r, c: (0, c))
    bspec = pl.BlockSpec((NT,), lambda r, c: (c,))
    hspec = pl.BlockSpec((BT, D), lambda r, c: (r, 0))
    gspec = pl.BlockSpec((BT, 1), lambda r, c: (r, 0))
    return pl.pallas_call(
        _heads_body,
        grid=grid,
        in_specs=[hspec, hspec, gspec, gspec,
                  wspec, bspec, wspec, bspec, wspec, bspec],
        out_specs=pl.BlockSpec((BT, NT), lambda r, c: (r, c)),
        out_shape=jax.ShapeDtypeStruct((B, N), jnp.float32),
    )(h1, h2, gate1, gate2, we1, be1, we2, be2, w3, b3)


def kernel(X, W1, b1, g1, We1, be1, W2, b2, g2, We2, be2, W3, b3):
    O = be1.shape[0]
    Opad = (O + NT - 1) // NT * NT
    pad = Opad - O
    we1 = jnp.pad(We1, ((0, 0), (0, pad)))
    we2 = jnp.pad(We2, ((0, 0), (0, pad)))
    w3 = jnp.pad(W3, ((0, 0), (0, pad)))
    be1p = jnp.pad(be1, (0, pad))
    be2p = jnp.pad(be2, (0, pad))
    b3p = jnp.pad(b3, (0, pad))

    h1, gate1 = _backbone_block(X, W1, b1, g1)
    h2, gate2 = _backbone_block(h1, W2, b2, g2)
    y = _heads(h1, h2, gate1, gate2, we1, be1p, we2, be2p, w3, b3p)
    return y[:, :O]


# speedup vs baseline: 1.9425x; 1.9425x over previous
import jax
import jax.numpy as jnp
from jax.experimental import pallas as pl

BT = 256


def _block_body(x_ref, w_ref, b_ref, g_ref, h_ref, gate_ref):
    h = jnp.dot(x_ref[...], w_ref[...], preferred_element_type=jnp.float32)
    h = jnp.maximum(h + b_ref[...][None, :], 0.0)
    h_ref[...] = h
    gate_ref[...] = jnp.dot(h, g_ref[...], preferred_element_type=jnp.float32)


def _backbone_block(x, w, b, g):
    B, D = x.shape
    N = w.shape[1]
    return pl.pallas_call(
        _block_body,
        grid=(B // BT,),
        in_specs=[
            pl.BlockSpec((BT, D), lambda r: (r, 0)),
            pl.BlockSpec((D, N), lambda r: (0, 0)),
            pl.BlockSpec((N,), lambda r: (0,)),
            pl.BlockSpec((N, 1), lambda r: (0, 0)),
        ],
        out_specs=[
            pl.BlockSpec((BT, N), lambda r: (r, 0)),
            pl.BlockSpec((BT, 1), lambda r: (r, 0)),
        ],
        out_shape=[
            jax.ShapeDtypeStruct((B, N), jnp.float32),
            jax.ShapeDtypeStruct((B, 1), jnp.float32),
        ],
    )(x, w, b, g)


def _heads_body(h1_ref, h2_ref, g1_ref, g2_ref,
                we1_ref, be1_ref, we2_ref, be2_ref, w3_ref, b3_ref, y_ref):
    take1 = g1_ref[...] > 0.0
    take2 = jnp.logical_and(jnp.logical_not(take1), g2_ref[...] > 0.0)
    y1 = jnp.dot(h1_ref[...], we1_ref[...], preferred_element_type=jnp.float32)
    y2 = jnp.dot(h2_ref[...], we2_ref[...], preferred_element_type=jnp.float32)
    y3 = jnp.dot(h2_ref[...], w3_ref[...], preferred_element_type=jnp.float32)
    b = jnp.where(take1, be1_ref[...],
                  jnp.where(take2, be2_ref[...], b3_ref[...]))
    y_ref[...] = jnp.where(take1, y1, jnp.where(take2, y2, y3)) + b


def _heads(h1, h2, gate1, gate2, we1, be1, we2, be2, w3, b3):
    B, D = h1.shape
    N = we1.shape[1]
    wspec = pl.BlockSpec((D, N), lambda r: (0, 0))
    bspec = pl.BlockSpec((1, N), lambda r: (0, 0))
    hspec = pl.BlockSpec((BT, D), lambda r: (r, 0))
    gspec = pl.BlockSpec((BT, 1), lambda r: (r, 0))
    return pl.pallas_call(
        _heads_body,
        grid=(B // BT,),
        in_specs=[hspec, hspec, gspec, gspec,
                  wspec, bspec, wspec, bspec, wspec, bspec],
        out_specs=pl.BlockSpec((BT, N), lambda r: (r, 0)),
        out_shape=jax.ShapeDtypeStruct((B, N), jnp.float32),
    )(h1, h2, gate1, gate2, we1, be1, we2, be2, w3, b3)


def kernel(X, W1, b1, g1, We1, be1, W2, b2, g2, We2, be2, W3, b3):
    O = be1.shape[0]
    Opad = 1024
    pad = Opad - O
    we1 = jnp.pad(We1, ((0, 0), (0, pad)))
    we2 = jnp.pad(We2, ((0, 0), (0, pad)))
    w3 = jnp.pad(W3, ((0, 0), (0, pad)))
    be1p = jnp.pad(be1, (0, pad)).reshape(1, Opad)
    be2p = jnp.pad(be2, (0, pad)).reshape(1, Opad)
    b3p = jnp.pad(b3, (0, pad)).reshape(1, Opad)

    h1, gate1 = _backbone_block(X, W1, b1, g1)
    h2, gate2 = _backbone_block(h1, W2, b2, g2)
    y = _heads(h1, h2, gate1, gate2, we1, be1p, we2, be2p, w3, b3p)
    return y[:, :O]
